# baseline (speedup 1.0000x reference)
import functools

import jax
import jax.numpy as jnp
from jax.experimental import pallas as pl
from jax.experimental.pallas import tpu as pltpu
from jax.experimental.pallas import tpu_sc as plsc

_WIN = 128
_CHUNK = 512
_NWIN = _CHUNK // _WIN


def _gather_rows(weight, idx2d, n, d):
    mesh = plsc.VectorSubcoreMesh(core_axis_name="core",
                                  subcore_axis_name="subcore")

    @functools.partial(
        pl.kernel,
        out_type=jax.ShapeDtypeStruct((n, d), weight.dtype),
        mesh=mesh,
        scratch_types=[pltpu.SemaphoreType.DMA],
        compiler_params=pltpu.CompilerParams(use_tc_tiling_on_sc=False),
    )
    def gather_kernel(w_hbm, i_hbm, o_hbm, sem):
        def body(i_vmem, o_vmem):
            copies = [
                pltpu.async_copy(
                    w_hbm.at[i_vmem.at[0, pl.ds(j * _WIN, _WIN)]],
                    o_vmem.at[pl.ds(j * _WIN, _WIN)],
                    sem,
                )
                for j in range(_NWIN)
            ]
            for c in copies:
                c.wait()

        pltpu.emit_pipeline(
            body,
            grid=(n // _CHUNK,),
            in_specs=[pl.BlockSpec((1, _CHUNK), index_map=lambda i: (0, i))],
            out_specs=[pl.BlockSpec((_CHUNK, d), index_map=lambda i: (i, 0))],
            core_axis_name=("core", "subcore"),
            dimension_semantics=(pltpu.PARALLEL,),
        )(i_hbm, o_hbm)

    return gather_kernel(weight, idx2d)


def kernel(token_ids, weight):
    b, t = token_ids.shape
    n = b * t
    d = weight.shape[1]
    idx2d = token_ids.reshape(1, n).astype(jnp.int32)
    out = _gather_rows(weight, idx2d, n, d)
    return out.reshape(b, t, d)


# speedup vs baseline: 1.1170x; 1.1170x over previous
import functools

import jax
import jax.numpy as jnp
from jax.experimental import pallas as pl
from jax.experimental.pallas import tpu as pltpu
from jax.experimental.pallas import tpu_sc as plsc

_WIN = 128
_CHUNK = 128
_NWIN = _CHUNK // _WIN


def _gather_rows(weight, idx2d, n, d):
    mesh = plsc.VectorSubcoreMesh(core_axis_name="core",
                                  subcore_axis_name="subcore")

    @functools.partial(
        pl.kernel,
        out_type=jax.ShapeDtypeStruct((n, d), weight.dtype),
        mesh=mesh,
        scratch_types=[pltpu.SemaphoreType.DMA],
        compiler_params=pltpu.CompilerParams(use_tc_tiling_on_sc=False),
    )
    def gather_kernel(w_hbm, i_hbm, o_hbm, sem):
        def body(i_vmem, o_vmem):
            copies = [
                pltpu.async_copy(
                    w_hbm.at[i_vmem.at[0, pl.ds(j * _WIN, _WIN)]],
                    o_vmem.at[pl.ds(j * _WIN, _WIN)],
                    sem,
                )
                for j in range(_NWIN)
            ]
            for c in copies:
                c.wait()

        pltpu.emit_pipeline(
            body,
            grid=(n // _CHUNK,),
            in_specs=[pl.BlockSpec((1, _CHUNK), index_map=lambda i: (0, i))],
            out_specs=[pl.BlockSpec((_CHUNK, d), index_map=lambda i: (i, 0))],
            core_axis_name=("core", "subcore"),
            dimension_semantics=(pltpu.PARALLEL,),
        )(i_hbm, o_hbm)

    return gather_kernel(weight, idx2d)


def kernel(token_ids, weight):
    b, t = token_ids.shape
    n = b * t
    d = weight.shape[1]
    idx2d = token_ids.reshape(1, n).astype(jnp.int32)
    out = _gather_rows(weight, idx2d, n, d)
    return out.reshape(b, t, d)
